# baseline (speedup 1.0000x reference)
import functools
import jax
import jax.numpy as jnp
from jax import lax
from jax.experimental import pallas as pl
from jax.experimental.pallas import tpu as pltpu
from jax.experimental.pallas import tpu_sc as plsc

B = 4096
N = 1000
K = 9
NC = 2
NS = 16
NW = NC * NS
NSPLIT = 2
BH = B // NSPLIT
ROWS_PER_W = BH // NW
G = 16
NG = ROWS_PER_W // G
RIL = 16
NBLK = N // 16
LN2 = 0.6931471805599453


def _log_1_to_16(s):
    bits = lax.bitcast_convert_type(s, jnp.int32)
    e = jnp.float32(1.0) * ((bits >> 23) - 127)
    m = lax.bitcast_convert_type(
        (bits & jnp.int32(0x007FFFFF)) | jnp.int32(0x3F800000), jnp.float32)
    u = (m - 1.0) / (m + 1.0)
    u2 = u * u
    p = 2.0 * u * (1.0 + u2 * (1.0 / 3.0 + u2 * (1.0 / 5.0
                   + u2 * (1.0 / 7.0 + u2 * (1.0 / 9.0)))))
    return e * LN2 + p


def _mmcl_body(lg_hbm, tg_hbm, out_hbm, buf0, buf1, tgts, res, ovec,
               sem0, sem1):
    wid = lax.axis_index("s") * NC + lax.axis_index("c")
    row0 = wid * ROWS_PER_W
    lanes = lax.iota(jnp.int32, 16)

    pltpu.sync_copy(tg_hbm.at[pl.ds(row0 * 1, ROWS_PER_W)], tgts)

    sems = [sem0, sem1]
    bufs = [buf0, buf1]
    acc = jnp.zeros((16,), jnp.float32)
    neg_inf = jnp.full((16,), -jnp.inf, jnp.float32)
    tail_cv = jnp.full((16,), 984, jnp.int32) + lanes
    tail_mask = lanes < 8

    pending = pltpu.async_copy(
        lg_hbm.at[pl.ds(row0, G)], bufs[0], sems[0])
    for g in range(NG):
        cur = g % 2
        nxt = (g + 1) % 2
        pending.wait()
        if g + 1 < NG:
            pending = pltpu.async_copy(
                lg_hbm.at[pl.ds(row0 + (g + 1) * G, G)],
                bufs[nxt], sems[nxt])

        bufv = bufs[cur]
        tgt16 = tgts[pl.ds(g * G, 16)]

        pos = plsc.load_gather(bufv, [lanes, tgt16])
        plsc.store_scatter(bufv, [lanes, tgt16], neg_inf)

        for batch in range(G // RIL):
            rows = [batch * RIL + s for s in range(RIL)]
            rowvecs = [jnp.full((16,), r, jnp.int32) for r in rows]
            cv0 = lanes
            Ts0 = tuple(neg_inf for _ in range(RIL))

            def body(i, carry, bufv=bufv, rowvecs=rowvecs):
                cv, Ts = carry
                Ts2 = []
                for s in range(RIL):
                    v = plsc.load_gather(bufv, [rowvecs[s], cv])
                    vd, _ = plsc.sort_key_val(v, v, descending=True)
                    m = jnp.maximum(Ts[s], vd)
                    Ts2.append(lax.sort(m))
                return cv + 16, tuple(Ts2)

            _, Ts = lax.fori_loop(0, NBLK, body, (cv0, Ts0))

            for s in range(RIL):
                v = plsc.load_gather(bufv, [rowvecs[s], tail_cv])
                v = jnp.where(tail_mask, neg_inf, v)
                vd, _ = plsc.sort_key_val(v, v, descending=True)
                m = jnp.maximum(Ts[s], vd)
                t_fin = lax.sort(m)
                res[pl.ds(rows[s] * 16, 16)] = t_fin

        posx = pos * 10.0
        top1 = plsc.load_gather(res, [lanes * 16 + 15])
        mx = jnp.maximum(top1 * 10.0, posx)
        s = 2.0 * jnp.exp(posx - mx)
        for i in range(16 - K, 16):
            vi = plsc.load_gather(res, [lanes * 16 + i])
            s = s + jnp.exp(vi * 10.0 - mx)
        loss = _log_1_to_16(s) + mx - posx
        acc = acc + loss * (1.0 / B)

    ovec[...] = acc
    pltpu.sync_copy(ovec, out_hbm.at[pl.ds(wid * 16, 16)])


@jax.jit
def _mmcl(logits, targets):
    mesh = plsc.VectorSubcoreMesh(core_axis_name="c", subcore_axis_name="s")
    sc_call = pl.kernel(
        _mmcl_body,
        mesh=mesh,
        compiler_params=pltpu.CompilerParams(
            needs_layout_passes=False, use_tc_tiling_on_sc=True),
        out_type=jax.ShapeDtypeStruct((NW * 16,), jnp.float32),
        scratch_types=[
            pltpu.VMEM((G, N), jnp.float32),
            pltpu.VMEM((G, N), jnp.float32),
            pltpu.VMEM((ROWS_PER_W,), jnp.int32),
            pltpu.VMEM((G * 16,), jnp.float32),
            pltpu.VMEM((16,), jnp.float32),
            pltpu.SemaphoreType.DMA,
            pltpu.SemaphoreType.DMA,
        ],
    )
    acc = jnp.float32(0.0)
    for h in range(NSPLIT):
        part = sc_call(logits[h * BH:(h + 1) * BH],
                       targets[h * BH:(h + 1) * BH])
        acc = acc + jnp.sum(part)
    return acc


def kernel(logits, targets):
    targets = targets.astype(jnp.int32)
    return _mmcl(logits, targets)


# speedup vs baseline: 1.3129x; 1.3129x over previous
import jax
import jax.numpy as jnp
from jax import lax
from jax.experimental import pallas as pl
from jax.experimental.pallas import tpu as pltpu
from jax.experimental.pallas import tpu_sc as plsc

B = 4096
N = 1000
K = 9
NC = 2
NS = 16
NW = NC * NS
ROWS_PER_W = B // NW
G = 16
NG = ROWS_PER_W // G
RIL = 16
NBLK = N // 16
LN2 = 0.6931471805599453


def _log_1_to_16(s):
    bits = lax.bitcast_convert_type(s, jnp.int32)
    e = jnp.float32(1.0) * ((bits >> 23) - 127)
    m = lax.bitcast_convert_type(
        (bits & jnp.int32(0x007FFFFF)) | jnp.int32(0x3F800000), jnp.float32)
    u = (m - 1.0) / (m + 1.0)
    u2 = u * u
    p = 2.0 * u * (1.0 + u2 * (1.0 / 3.0 + u2 * (1.0 / 5.0
                   + u2 * (1.0 / 7.0 + u2 * (1.0 / 9.0)))))
    return e * LN2 + p


def _mmcl_body(lg_hbm, tg_hbm, out_hbm, buf0, buf1, tgts, res, ovec,
               sem0, sem1):
    wid = lax.axis_index("s") * NC + lax.axis_index("c")
    row0 = wid * ROWS_PER_W
    lanes = lax.iota(jnp.int32, 16)

    pltpu.sync_copy(tg_hbm.at[pl.ds(row0 * 1, ROWS_PER_W)], tgts)

    sems = [sem0, sem1]
    bufs = [buf0, buf1]
    acc = jnp.zeros((16,), jnp.float32)
    neg_inf = jnp.full((16,), -jnp.inf, jnp.float32)
    tail_cv = jnp.full((16,), 984, jnp.int32) + lanes
    tail_mask = lanes < 8

    pending = pltpu.async_copy(
        lg_hbm.at[pl.ds(row0, G)], bufs[0], sems[0])
    for g in range(NG):
        cur = g % 2
        nxt = (g + 1) % 2
        pending.wait()
        if g + 1 < NG:
            pending = pltpu.async_copy(
                lg_hbm.at[pl.ds(row0 + (g + 1) * G, G)],
                bufs[nxt], sems[nxt])

        bufv = bufs[cur]
        tgt16 = tgts[pl.ds(g * G, 16)]

        pos = plsc.load_gather(bufv, [lanes, tgt16])
        plsc.store_scatter(bufv, [lanes, tgt16], neg_inf)

        for batch in range(G // RIL):
            rows = [batch * RIL + s for s in range(RIL)]
            rowvecs = [jnp.full((16,), r, jnp.int32) for r in rows]
            cv0 = lanes
            Ts0 = tuple(neg_inf for _ in range(RIL))

            def body(i, carry, bufv=bufv, rowvecs=rowvecs):
                cv, Ts = carry
                Ts2 = []
                for s in range(RIL):
                    v = plsc.load_gather(bufv, [rowvecs[s], cv])
                    vd, _ = plsc.sort_key_val(v, v, descending=True)
                    m = jnp.maximum(Ts[s], vd)
                    Ts2.append(lax.sort(m))
                return cv + 16, tuple(Ts2)

            _, Ts = lax.fori_loop(0, NBLK, body, (cv0, Ts0))

            for s in range(RIL):
                v = plsc.load_gather(bufv, [rowvecs[s], tail_cv])
                v = jnp.where(tail_mask, neg_inf, v)
                vd, _ = plsc.sort_key_val(v, v, descending=True)
                m = jnp.maximum(Ts[s], vd)
                t_fin = lax.sort(m)
                res[pl.ds(rows[s] * 16, 16)] = t_fin

        posx = pos * 10.0
        top1 = plsc.load_gather(res, [lanes * 16 + 15])
        mx = jnp.maximum(top1 * 10.0, posx)
        s = 2.0 * jnp.exp(posx - mx)
        for i in range(16 - K, 16):
            vi = plsc.load_gather(res, [lanes * 16 + i])
            s = s + jnp.exp(vi * 10.0 - mx)
        loss = _log_1_to_16(s) + mx - posx
        acc = acc + loss * (1.0 / B)

    ovec[...] = acc
    pltpu.sync_copy(ovec, out_hbm.at[pl.ds(wid * 16, 16)])


@jax.jit
def _mmcl(logits, targets):
    mesh = plsc.VectorSubcoreMesh(core_axis_name="c", subcore_axis_name="s")
    partials = pl.kernel(
        _mmcl_body,
        mesh=mesh,
        compiler_params=pltpu.CompilerParams(
            needs_layout_passes=False, use_tc_tiling_on_sc=True),
        out_type=jax.ShapeDtypeStruct((NW * 16,), jnp.float32),
        scratch_types=[
            pltpu.VMEM((G, N), jnp.float32),
            pltpu.VMEM((G, N), jnp.float32),
            pltpu.VMEM((ROWS_PER_W,), jnp.int32),
            pltpu.VMEM((G * 16,), jnp.float32),
            pltpu.VMEM((16,), jnp.float32),
            pltpu.SemaphoreType.DMA,
            pltpu.SemaphoreType.DMA,
        ],
    )(logits, targets)
    return jnp.sum(partials)


def kernel(logits, targets):
    targets = targets.astype(jnp.int32)
    return _mmcl(logits, targets)
